# baseline (speedup 1.0000x reference)
import functools

import jax
import jax.numpy as jnp
from jax import lax
from jax.experimental import pallas as pl
from jax.experimental.pallas import tpu as pltpu
from jax.experimental.pallas import tpu_sc as plsc

_NPAD = 10112
_MP = 40960
_NW = 32
_QPW = _MP // _NW
_CH = 40
_NCH = _QPW // _CH
_BQ = 400


def _ln(x, g, b):
    mu = jnp.mean(x, axis=-1, keepdims=True)
    var = jnp.mean((x - mu) ** 2, axis=-1, keepdims=True)
    return (x - mu) / jnp.sqrt(var + 1e-5) * g + b


def _lnmm_body(x_ref, g_ref, b_ref, w_ref, bias_ref, o_ref):
    xn = _ln(x_ref[...], g_ref[...], b_ref[...])
    o_ref[...] = (
        jnp.dot(xn, w_ref[...], preferred_element_type=jnp.float32)
        + bias_ref[...]
    )


def _ln_matmul(x, g, b, w, bias, block_rows):
    rows, cin = x.shape
    cout = w.shape[1]
    return pl.pallas_call(
        _lnmm_body,
        grid=(rows // block_rows,),
        in_specs=[
            pl.BlockSpec((block_rows, cin), lambda i: (i, 0)),
            pl.BlockSpec((cin,), lambda i: (0,)),
            pl.BlockSpec((cin,), lambda i: (0,)),
            pl.BlockSpec((cin, cout), lambda i: (0, 0)),
            pl.BlockSpec((cout,), lambda i: (0,)),
        ],
        out_specs=pl.BlockSpec((block_rows, cout), lambda i: (i, 0)),
        out_shape=jax.ShapeDtypeStruct((rows, cout), jnp.float32),
    )(x, g, b, w, bias)


def _knn_body(x_ref, g_ref, b_ref, w_ref, bias_ref, q_ref, ct_ref, cf_ref,
              b1_ref, i0_ref, i1_ref, i2_ref, w0_ref, w1_ref, w2_ref,
              d2_ref):
    xn = _ln(x_ref[...], g_ref[...], b_ref[...])
    b1_ref[...] = (
        jnp.dot(xn, w_ref[...], preferred_element_type=jnp.float32)
        + bias_ref[...]
    )

    q = q_ref[...]
    qx, qy, qz = q[:, 0:1], q[:, 1:2], q[:, 2:3]
    cx = ct_ref[0:1, :]
    cy = ct_ref[1:2, :]
    cz = ct_ref[2:3, :]
    dx = qx - cx
    dy = qy - cy
    dz = qz - cz
    d2v = dx * dx + dy * dy + dz * dz
    d2_ref[...] = d2v
    colsf = cf_ref[0:1, :]

    picks = []
    for k in range(3):
        d2 = d2v if k == 0 else d2_ref[...]
        m = jnp.min(d2, axis=1, keepdims=True)
        ik = jnp.min(jnp.where(d2 == m, colsf, jnp.float32(1e9)),
                     axis=1, keepdims=True)
        if k < 2:
            d2_ref[...] = jnp.where(colsf == ik, jnp.float32(jnp.inf), d2)
        picks.append((m, ik))

    ws = [1.0 / (jnp.sqrt(jnp.maximum(m, 1e-12)) + 1e-8) for m, _ in picks]
    s = ws[0] + ws[1] + ws[2]
    i0_ref[...] = picks[0][1].astype(jnp.int32)
    i1_ref[...] = picks[1][1].astype(jnp.int32)
    i2_ref[...] = picks[2][1].astype(jnp.int32)
    w0_ref[...] = ws[0] / s
    w1_ref[...] = ws[1] / s
    w2_ref[...] = ws[2] / s


def _knn_branch1(skip_feats, ln1_g, ln1_b, W1, b1, q, ct, cf):
    m = q.shape[0]
    cout = W1.shape[1]
    out_shape = [jax.ShapeDtypeStruct((m, cout), jnp.float32)] + \
        [jax.ShapeDtypeStruct((m, 1), jnp.int32)] * 3 + \
        [jax.ShapeDtypeStruct((m, 1), jnp.float32)] * 3
    o_spec = pl.BlockSpec((_BQ, 1), lambda i: (i, 0))
    return pl.pallas_call(
        _knn_body,
        grid=(m // _BQ,),
        in_specs=[
            pl.BlockSpec((_BQ, cout), lambda i: (i, 0)),
            pl.BlockSpec((cout,), lambda i: (0,)),
            pl.BlockSpec((cout,), lambda i: (0,)),
            pl.BlockSpec((cout, cout), lambda i: (0, 0)),
            pl.BlockSpec((cout,), lambda i: (0,)),
            pl.BlockSpec((_BQ, 3), lambda i: (i, 0)),
            pl.BlockSpec((3, _NPAD), lambda i: (0, 0)),
            pl.BlockSpec((1, _NPAD), lambda i: (0, 0)),
        ],
        out_specs=[pl.BlockSpec((_BQ, cout), lambda i: (i, 0))] + [o_spec] * 6,
        out_shape=out_shape,
        scratch_shapes=[pltpu.VMEM((_BQ, _NPAD), jnp.float32)],
    )(skip_feats, ln1_g, ln1_b, W1, b1, q, ct, cf)


def _sc_gather_combine(feats2, b1, i0, i1, i2, w0, w1, w2, m_valid):
    cout = feats2.shape[1]
    mesh = plsc.VectorSubcoreMesh(core_axis_name="c", subcore_axis_name="s")

    @functools.partial(
        pl.kernel,
        out_type=jax.ShapeDtypeStruct((_MP, cout), jnp.float32),
        mesh=mesh,
        compiler_params=pltpu.CompilerParams(needs_layout_passes=False),
        scratch_types=[
            pltpu.VMEM((_QPW,), jnp.int32),
            pltpu.VMEM((_QPW,), jnp.int32),
            pltpu.VMEM((_QPW,), jnp.int32),
            pltpu.VMEM((_QPW,), jnp.float32),
            pltpu.VMEM((_QPW,), jnp.float32),
            pltpu.VMEM((_QPW,), jnp.float32),
            pltpu.VMEM((2, _CH, cout), jnp.float32),
            pltpu.VMEM((2, _CH, cout), jnp.float32),
            pltpu.VMEM((2, _CH, cout), jnp.float32),
            pltpu.VMEM((_CH, cout), jnp.float32),
            pltpu.VMEM((_CH, cout), jnp.float32),
            pltpu.SemaphoreType.DMA,
            pltpu.SemaphoreType.DMA,
        ],
    )
    def body(f2_hbm, b1_hbm, i0_hbm, i1_hbm, i2_hbm, w0_hbm, w1_hbm, w2_hbm,
             out_hbm, i0_v, i1_v, i2_v, w0_v, w1_v, w2_v,
             r0_v, r1_v, r2_v, b1_v, o_v, sA, sB):
        cid = lax.axis_index("c")
        sid = lax.axis_index("s")
        wid = sid * 2 + cid
        qb = pl.multiple_of(wid * _QPW, 8)
        pltpu.sync_copy(i0_hbm.at[pl.ds(qb, _QPW)], i0_v)
        pltpu.sync_copy(i1_hbm.at[pl.ds(qb, _QPW)], i1_v)
        pltpu.sync_copy(i2_hbm.at[pl.ds(qb, _QPW)], i2_v)
        pltpu.sync_copy(w0_hbm.at[pl.ds(qb, _QPW)], w0_v)
        pltpu.sync_copy(w1_hbm.at[pl.ds(qb, _QPW)], w1_v)
        pltpu.sync_copy(w2_hbm.at[pl.ds(qb, _QPW)], w2_v)

        sems = (sA, sB)

        def issue(g, buf, sem):
            off = pl.multiple_of(g * _CH, 8)
            pltpu.async_copy(f2_hbm.at[i0_v.at[pl.ds(off, _CH)]],
                             r0_v.at[buf], sem)
            pltpu.async_copy(f2_hbm.at[i1_v.at[pl.ds(off, _CH)]],
                             r1_v.at[buf], sem)
            pltpu.async_copy(f2_hbm.at[i2_v.at[pl.ds(off, _CH)]],
                             r2_v.at[buf], sem)

        def drain(buf, sem):
            for r_v in (r0_v, r1_v, r2_v):
                pltpu.make_async_copy(f2_hbm.at[pl.ds(0, _CH)],
                                      r_v.at[buf], sem).wait()

        issue(0, 0, sA)

        def outer(g2, carry):
            for b in range(2):
                g = 2 * g2 + b
                off = pl.multiple_of(g * _CH, 8)

                @pl.when(g + 1 < _NCH)
                def _():
                    issue(g + 1, 1 - b, sems[1 - b])

                drain(b, sems[b])

                @pl.when(qb + off < m_valid)
                def _():
                    pltpu.sync_copy(b1_hbm.at[pl.ds(qb + off, _CH)], b1_v)

                    def qloop(c, carry2):
                        oidx = jnp.full((16,), off + c, jnp.int32)
                        a0 = plsc.load_gather(w0_v, [oidx])
                        a1 = plsc.load_gather(w1_v, [oidx])
                        a2 = plsc.load_gather(w2_v, [oidx])
                        for v in range(cout // 16):
                            sl = pl.ds(v * 16, 16)
                            o_v[c, sl] = (
                                b1_v[c, sl] + a0 * r0_v[b, c, sl]
                                + a1 * r1_v[b, c, sl] + a2 * r2_v[b, c, sl])
                        return carry2

                    lax.fori_loop(0, _CH, qloop, 0)
                    pltpu.sync_copy(o_v, out_hbm.at[pl.ds(qb + off, _CH)])

            return carry

        lax.fori_loop(0, _NCH // 2, outer, 0)

    return body(feats2, b1, i0, i1, i2, w0, w1, w2)


def kernel(feats, coords, offset, skip_feats, skip_coords, skip_offset,
           ln1_g, ln1_b, W1, b1, ln2_g, ln2_b, W2, b2):
    m = skip_coords.shape[0]
    n = coords.shape[0]

    feats2 = _ln_matmul(feats, ln2_g, ln2_b, W2, b2, block_rows=2000)

    ct = jnp.pad(coords.T, ((0, 0), (0, _NPAD - n)), constant_values=100.0)
    cf = jnp.arange(_NPAD, dtype=jnp.float32).reshape(1, _NPAD)
    branch1, i0, i1, i2, w0, w1, w2 = _knn_branch1(
        skip_feats, ln1_g, ln1_b, W1, b1, skip_coords, ct, cf)

    def flat_pad(x):
        return jnp.pad(x.reshape(-1), (0, _MP - m))

    out_full = _sc_gather_combine(
        feats2, branch1,
        flat_pad(i0), flat_pad(i1), flat_pad(i2),
        flat_pad(w0), flat_pad(w1), flat_pad(w2), m)
    return (out_full[:m], skip_coords, skip_offset)


# speedup vs baseline: 1.0940x; 1.0940x over previous
import functools

import jax
import jax.numpy as jnp
from jax import lax
from jax.experimental import pallas as pl
from jax.experimental.pallas import tpu as pltpu
from jax.experimental.pallas import tpu_sc as plsc

_NPAD = 10112
_MP = 40960
_NW = 32
_QPW = _MP // _NW
_CH = 80
_NCH = _QPW // _CH
_BQ = 400


def _ln(x, g, b):
    mu = jnp.mean(x, axis=-1, keepdims=True)
    var = jnp.mean((x - mu) ** 2, axis=-1, keepdims=True)
    return (x - mu) / jnp.sqrt(var + 1e-5) * g + b


def _lnmm_body(x_ref, g_ref, b_ref, w_ref, bias_ref, o_ref):
    xn = _ln(x_ref[...], g_ref[...], b_ref[...])
    o_ref[...] = (
        jnp.dot(xn, w_ref[...], preferred_element_type=jnp.float32)
        + bias_ref[...]
    )


def _ln_matmul(x, g, b, w, bias, block_rows):
    rows, cin = x.shape
    cout = w.shape[1]
    return pl.pallas_call(
        _lnmm_body,
        grid=(rows // block_rows,),
        in_specs=[
            pl.BlockSpec((block_rows, cin), lambda i: (i, 0)),
            pl.BlockSpec((cin,), lambda i: (0,)),
            pl.BlockSpec((cin,), lambda i: (0,)),
            pl.BlockSpec((cin, cout), lambda i: (0, 0)),
            pl.BlockSpec((cout,), lambda i: (0,)),
        ],
        out_specs=pl.BlockSpec((block_rows, cout), lambda i: (i, 0)),
        out_shape=jax.ShapeDtypeStruct((rows, cout), jnp.float32),
    )(x, g, b, w, bias)


def _knn_body(x_ref, g_ref, b_ref, w_ref, bias_ref, q_ref, ct_ref,
              b1_ref, i0_ref, i1_ref, i2_ref, w0_ref, w1_ref, w2_ref,
              d2_ref):
    xn = _ln(x_ref[...], g_ref[...], b_ref[...])
    b1_ref[...] = (
        jnp.dot(xn, w_ref[...], preferred_element_type=jnp.float32)
        + bias_ref[...]
    )

    q = q_ref[...]
    qx, qy, qz = q[:, 0:1], q[:, 1:2], q[:, 2:3]
    cx = ct_ref[0:1, :]
    cy = ct_ref[1:2, :]
    cz = ct_ref[2:3, :]
    dx = qx - cx
    dy = qy - cy
    dz = qz - cz
    d2_ref[...] = dx * dx + dy * dy + dz * dz
    colsf = lax.broadcasted_iota(jnp.int32, (_BQ, _NPAD), 1).astype(jnp.float32)

    picks = []
    for k in range(3):
        d2 = d2_ref[...]
        m = jnp.min(d2, axis=1, keepdims=True)
        ik = jnp.min(jnp.where(d2 == m, colsf, jnp.float32(1e9)),
                     axis=1, keepdims=True)
        if k < 2:
            d2_ref[...] = jnp.where(colsf == ik, jnp.float32(jnp.inf), d2)
        picks.append((m, ik))

    ws = [1.0 / (jnp.sqrt(jnp.maximum(m, 1e-12)) + 1e-8) for m, _ in picks]
    s = ws[0] + ws[1] + ws[2]
    i0_ref[...] = picks[0][1].astype(jnp.int32)
    i1_ref[...] = picks[1][1].astype(jnp.int32)
    i2_ref[...] = picks[2][1].astype(jnp.int32)
    w0_ref[...] = ws[0] / s
    w1_ref[...] = ws[1] / s
    w2_ref[...] = ws[2] / s


def _knn_branch1(skip_feats, ln1_g, ln1_b, W1, b1, q, ct):
    m = q.shape[0]
    cout = W1.shape[1]
    out_shape = [jax.ShapeDtypeStruct((m, cout), jnp.float32)] + \
        [jax.ShapeDtypeStruct((m, 1), jnp.int32)] * 3 + \
        [jax.ShapeDtypeStruct((m, 1), jnp.float32)] * 3
    o_spec = pl.BlockSpec((_BQ, 1), lambda i: (i, 0))
    return pl.pallas_call(
        _knn_body,
        grid=(m // _BQ,),
        in_specs=[
            pl.BlockSpec((_BQ, cout), lambda i: (i, 0)),
            pl.BlockSpec((cout,), lambda i: (0,)),
            pl.BlockSpec((cout,), lambda i: (0,)),
            pl.BlockSpec((cout, cout), lambda i: (0, 0)),
            pl.BlockSpec((cout,), lambda i: (0,)),
            pl.BlockSpec((_BQ, 3), lambda i: (i, 0)),
            pl.BlockSpec((3, _NPAD), lambda i: (0, 0)),
        ],
        out_specs=[pl.BlockSpec((_BQ, cout), lambda i: (i, 0))] + [o_spec] * 6,
        out_shape=out_shape,
        scratch_shapes=[pltpu.VMEM((_BQ, _NPAD), jnp.float32)],
    )(skip_feats, ln1_g, ln1_b, W1, b1, q, ct)


def _sc_gather_combine(feats2, b1, i0, i1, i2, w0, w1, w2, m_valid):
    cout = feats2.shape[1]
    mesh = plsc.VectorSubcoreMesh(core_axis_name="c", subcore_axis_name="s")

    @functools.partial(
        pl.kernel,
        out_type=jax.ShapeDtypeStruct((_MP, cout), jnp.float32),
        mesh=mesh,
        compiler_params=pltpu.CompilerParams(needs_layout_passes=False),
        scratch_types=[
            pltpu.VMEM((_QPW,), jnp.int32),
            pltpu.VMEM((_QPW,), jnp.int32),
            pltpu.VMEM((_QPW,), jnp.int32),
            pltpu.VMEM((_QPW,), jnp.float32),
            pltpu.VMEM((_QPW,), jnp.float32),
            pltpu.VMEM((_QPW,), jnp.float32),
            pltpu.VMEM((_CH, cout), jnp.float32),
            pltpu.VMEM((_CH, cout), jnp.float32),
            pltpu.VMEM((_CH, cout), jnp.float32),
            pltpu.VMEM((_CH, cout), jnp.float32),
            pltpu.VMEM((_CH, cout), jnp.float32),
            pltpu.SemaphoreType.DMA,
            pltpu.SemaphoreType.DMA,
            pltpu.SemaphoreType.DMA,
        ],
    )
    def body(f2_hbm, b1_hbm, i0_hbm, i1_hbm, i2_hbm, w0_hbm, w1_hbm, w2_hbm,
             out_hbm, i0_v, i1_v, i2_v, w0_v, w1_v, w2_v,
             r0_v, r1_v, r2_v, b1_v, o_v, s0, s1, s2):
        cid = lax.axis_index("c")
        sid = lax.axis_index("s")
        wid = sid * 2 + cid
        qb = pl.multiple_of(wid * _QPW, 8)
        pltpu.sync_copy(i0_hbm.at[pl.ds(qb, _QPW)], i0_v)
        pltpu.sync_copy(i1_hbm.at[pl.ds(qb, _QPW)], i1_v)
        pltpu.sync_copy(i2_hbm.at[pl.ds(qb, _QPW)], i2_v)
        pltpu.sync_copy(w0_hbm.at[pl.ds(qb, _QPW)], w0_v)
        pltpu.sync_copy(w1_hbm.at[pl.ds(qb, _QPW)], w1_v)
        pltpu.sync_copy(w2_hbm.at[pl.ds(qb, _QPW)], w2_v)

        def chunk(g, carry):
            off = pl.multiple_of(g * _CH, 8)

            @pl.when(qb + off < m_valid)
            def _():
                cp0 = pltpu.async_copy(
                    f2_hbm.at[i0_v.at[pl.ds(off, _CH)]], r0_v, s0)
                cp1 = pltpu.async_copy(
                    f2_hbm.at[i1_v.at[pl.ds(off, _CH)]], r1_v, s1)
                cp2 = pltpu.async_copy(
                    f2_hbm.at[i2_v.at[pl.ds(off, _CH)]], r2_v, s2)
                pltpu.sync_copy(b1_hbm.at[pl.ds(qb + off, _CH)], b1_v)
                cp0.wait()
                cp1.wait()
                cp2.wait()

                def qloop(c, carry2):
                    oidx = jnp.full((16,), off + c, jnp.int32)
                    a0 = plsc.load_gather(w0_v, [oidx])
                    a1 = plsc.load_gather(w1_v, [oidx])
                    a2 = plsc.load_gather(w2_v, [oidx])
                    for v in range(cout // 16):
                        sl = pl.ds(v * 16, 16)
                        o_v[c, sl] = (b1_v[c, sl] + a0 * r0_v[c, sl]
                                      + a1 * r1_v[c, sl] + a2 * r2_v[c, sl])
                    return carry2

                lax.fori_loop(0, _CH, qloop, 0)
                pltpu.sync_copy(o_v, out_hbm.at[pl.ds(qb + off, _CH)])

            return carry

        lax.fori_loop(0, _NCH, chunk, 0)

    return body(feats2, b1, i0, i1, i2, w0, w1, w2)


def kernel(feats, coords, offset, skip_feats, skip_coords, skip_offset,
           ln1_g, ln1_b, W1, b1, ln2_g, ln2_b, W2, b2):
    m = skip_coords.shape[0]
    n = coords.shape[0]

    feats2 = _ln_matmul(feats, ln2_g, ln2_b, W2, b2, block_rows=2000)

    ct = jnp.pad(coords.T, ((0, 0), (0, _NPAD - n)), constant_values=100.0)
    branch1, i0, i1, i2, w0, w1, w2 = _knn_branch1(
        skip_feats, ln1_g, ln1_b, W1, b1, skip_coords, ct)

    def flat_pad(x):
        return jnp.pad(x.reshape(-1), (0, _MP - m))

    out_full = _sc_gather_combine(
        feats2, branch1,
        flat_pad(i0), flat_pad(i1), flat_pad(i2),
        flat_pad(w0), flat_pad(w1), flat_pad(w2), m)
    return (out_full[:m], skip_coords, skip_offset)
